# baseline (speedup 1.0000x reference)
import functools

import jax
import jax.numpy as jnp
from jax import lax
from jax.experimental import pallas as pl
from jax.experimental.pallas import tpu as pltpu
from jax.experimental.pallas import tpu_sc as plsc

_P = 3
_DIM = 3
_L = 16
_S = 4


def kernel(ctrl_pts, Nu, uspan):
    B, K, D1 = ctrl_pts.shape
    OUT = uspan.shape[0]
    KD = K * D1
    info = plsc.get_sparse_core_info()
    NC = info.num_cores
    NW = NC * info.num_subcores
    rows_per = B // NW
    NT = OUT // _L
    nchunks = rows_per // _S

    ctrl_flat = ctrl_pts
    CW = _S * KD
    OW = _S * OUT * _DIM
    OD = OUT * _DIM

    mesh = plsc.VectorSubcoreMesh(core_axis_name="c", subcore_axis_name="s")

    @functools.partial(
        pl.kernel,
        mesh=mesh,
        compiler_params=pltpu.CompilerParams(needs_layout_passes=False,
                                             use_tc_tiling_on_sc=False),
        out_type=jax.ShapeDtypeStruct((B, OUT, _DIM), jnp.float32),
        scratch_types=[
            pltpu.VMEM((_S, K, D1), jnp.float32),
            pltpu.VMEM((_S, K, D1), jnp.float32),
            pltpu.VMEM((_S, OUT, _DIM), jnp.float32),
            pltpu.VMEM((_S, OUT, _DIM), jnp.float32),
            pltpu.VMEM((OUT,), jnp.int32),
            pltpu.VMEM((OUT, _P + 1), jnp.float32),
            pltpu.VMEM((OUT,), jnp.int32),
            pltpu.VMEM((OUT,), jnp.int32),
            pltpu.VMEM(((_P + 1) * OUT,), jnp.float32),
            pltpu.SemaphoreType.DMA,
            pltpu.SemaphoreType.DMA,
            pltpu.SemaphoreType.DMA,
            pltpu.SemaphoreType.DMA,
        ],
    )
    def sc_kernel(ctrl_hbm, nu_hbm, usp_hbm, out_hbm,
                  cb0, cb1, ob0, ob1, usp_v, nu_v, g_v, uu_v, nuw_v,
                  si0, si1, so0, so1):
        wid = lax.axis_index("s") * NC + lax.axis_index("c")
        base_row = wid * rows_per

        pltpu.sync_copy(usp_hbm, usp_v)
        pltpu.sync_copy(nu_hbm, nu_v)

        lanes = lax.iota(jnp.int32, _L)

        def build(t, carry):
            u0 = t * _L
            usp = usp_v[pl.ds(u0, _L)]
            g_v[pl.ds(u0, _L)] = usp - _P
            uu_v[pl.ds(u0, _L)] = lanes + u0
            for p in range(_P + 1):
                psplat = jnp.full((_L,), p, jnp.int32)
                nuw_v[pl.ds(p * OUT + u0, _L)] = plsc.load_gather(
                    nu_v, [lanes + u0, psplat])
            return carry

        lax.fori_loop(0, NT, build, 0)

        dsplat = [jnp.full((_L,), d, jnp.int32) for d in range(D1)]

        def compute_chunk(cbuf, obuf):
            def grp(t, carry):
                u0 = t * _L
                g = g_v[pl.ds(u0, _L)]
                uu = uu_v[pl.ds(u0, _L)]
                nus = [nuw_v[pl.ds(p * OUT + u0, _L)] for p in range(_P + 1)]
                wrow = [g + p for p in range(_P + 1)]
                for r in range(_S):
                    rsplat = jnp.full((_L,), r, jnp.int32)
                    acc = [None] * D1
                    for p in range(_P + 1):
                        for d in range(D1):
                            w = plsc.load_gather(
                                cbuf, [rsplat, wrow[p], dsplat[d]])
                            term = w * nus[p]
                            acc[d] = term if acc[d] is None else acc[d] + term
                    inv = 1.0 / acc[_P]
                    for d in range(_DIM):
                        plsc.store_scatter(obuf, [rsplat, uu, dsplat[d]],
                                           acc[d] * inv)
                return carry

            lax.fori_loop(0, NT, grp, 0)

        def in_copy(c, buf, sem):
            pltpu.async_copy(ctrl_hbm.at[pl.ds(base_row + c * _S, _S)],
                             buf, sem)

        def in_wait(buf, sem):
            pltpu.make_async_copy(ctrl_hbm.at[pl.ds(base_row, _S)],
                                  buf, sem).wait()

        def out_copy(c, buf, sem):
            pltpu.async_copy(buf, out_hbm.at[pl.ds(base_row + c * _S, _S)],
                             sem)

        def out_wait(buf, sem):
            pltpu.make_async_copy(buf, out_hbm.at[pl.ds(base_row, _S)],
                                  sem).wait()

        in_copy(0, cb0, si0)

        def pair(i, carry):
            c0 = 2 * i

            @pl.when(c0 + 1 < nchunks)
            def _():
                in_copy(c0 + 1, cb1, si1)

            in_wait(cb0, si0)

            @pl.when(i > 0)
            def _():
                out_wait(ob0, so0)

            compute_chunk(cb0, ob0)
            out_copy(c0, ob0, so0)

            @pl.when(c0 + 2 < nchunks)
            def _():
                in_copy(c0 + 2, cb0, si0)

            in_wait(cb1, si1)

            @pl.when(i > 0)
            def _():
                out_wait(ob1, so1)

            compute_chunk(cb1, ob1)
            out_copy(c0 + 1, ob1, so1)
            return carry

        lax.fori_loop(0, nchunks // 2, pair, 0)
        out_wait(ob0, so0)
        out_wait(ob1, so1)

    return sc_kernel(ctrl_flat, Nu, uspan)


# speedup vs baseline: 1.1016x; 1.1016x over previous
import functools

import jax
import jax.numpy as jnp
from jax import lax
from jax.experimental import pallas as pl
from jax.experimental.pallas import tpu as pltpu
from jax.experimental.pallas import tpu_sc as plsc

_P = 3
_DIM = 3
_L = 16
_S = 4


def kernel(ctrl_pts, Nu, uspan):
    B, K, D1 = ctrl_pts.shape
    OUT = uspan.shape[0]
    KD = K * D1
    OD = OUT * _DIM
    info = plsc.get_sparse_core_info()
    NC = info.num_cores
    NW = NC * info.num_subcores
    rows_per = B // NW
    NT = OUT // _L
    CW = _S * KD
    OW = _S * OD
    nchunks = rows_per // _S

    ctrl_flat = ctrl_pts.reshape(B * KD // 128, 128)

    mesh = plsc.VectorSubcoreMesh(core_axis_name="c", subcore_axis_name="s")

    @functools.partial(
        pl.kernel,
        mesh=mesh,
        compiler_params=pltpu.CompilerParams(needs_layout_passes=False),
        out_type=jax.ShapeDtypeStruct((B * OD // 128, 128), jnp.float32),
        scratch_types=[
            pltpu.VMEM((CW // 128, 128), jnp.float32),
            pltpu.VMEM((CW // 128, 128), jnp.float32),
            pltpu.VMEM((OW // 128, 128), jnp.float32),
            pltpu.VMEM((OW // 128, 128), jnp.float32),
            pltpu.VMEM((OUT,), jnp.int32),
            pltpu.VMEM((OUT, _P + 1), jnp.float32),
            pltpu.VMEM((OUT,), jnp.int32),
            pltpu.VMEM((OUT,), jnp.int32),
            pltpu.VMEM(((_P + 1) * OUT,), jnp.float32),
            pltpu.SemaphoreType.DMA,
            pltpu.SemaphoreType.DMA,
            pltpu.SemaphoreType.DMA,
            pltpu.SemaphoreType.DMA,
        ],
    )
    def sc_kernel(ctrl_hbm, nu_hbm, usp_hbm, out_hbm,
                  cb0, cb1, ob0, ob1, usp_v, nu_v, g_v, uu_v, nuw_v,
                  si0, si1, so0, so1):
        wid = lax.axis_index("s") * NC + lax.axis_index("c")
        base_crow = wid * rows_per * KD // 128
        base_orow = wid * rows_per * OD // 128

        pltpu.sync_copy(usp_hbm, usp_v)
        pltpu.sync_copy(nu_hbm, nu_v)

        lanes = lax.iota(jnp.int32, _L)

        def build(t, carry):
            u0 = t * _L
            usp = usp_v[pl.ds(u0, _L)]
            g_v[pl.ds(u0, _L)] = (usp - _P) * D1
            uu_v[pl.ds(u0, _L)] = (lanes + u0) * _DIM
            for p in range(_P + 1):
                psplat = jnp.full((_L,), p, jnp.int32)
                nuw_v[pl.ds(p * OUT + u0, _L)] = plsc.load_gather(
                    nu_v, [lanes + u0, psplat])
            return carry

        lax.fori_loop(0, NT, build, 0)

        def compute_chunk(cbuf, obuf):
            def grp(t, carry):
                u0 = t * _L
                g4 = g_v[pl.ds(u0, _L)]
                uu = uu_v[pl.ds(u0, _L)]
                nus = [nuw_v[pl.ds(p * OUT + u0, _L)] for p in range(_P + 1)]
                nj = (_P + 1) * D1
                win = [g4 + j for j in range(nj)]
                winhi = [lax.shift_right_logical(w, 7) for w in win]
                winlo = [lax.bitwise_and(w, 127) for w in win]
                bflat = [uu + d for d in range(_DIM)]
                bhi = [lax.shift_right_logical(b, 7) for b in bflat]
                blo = [lax.bitwise_and(b, 127) for b in bflat]
                for r in range(_S):
                    acc = [None] * D1
                    for j in range(nj):
                        p, d = j // D1, j % D1
                        w = plsc.load_gather(
                            cbuf, [winhi[j] + r * (KD // 128), winlo[j]])
                        term = w * nus[p]
                        acc[d] = term if acc[d] is None else acc[d] + term
                    inv = 1.0 / acc[_P]
                    for d in range(_DIM):
                        plsc.store_scatter(
                            obuf, [bhi[d] + r * (OD // 128), blo[d]],
                            acc[d] * inv)
                return carry

            lax.fori_loop(0, NT, grp, 0)

        CR = CW // 128
        OR_ = OW // 128

        def in_copy(c, buf, sem):
            start = pl.multiple_of(base_crow + c * CR, 8)
            pltpu.async_copy(ctrl_hbm.at[pl.ds(start, CR)], buf, sem)

        def in_wait(buf, sem):
            start = pl.multiple_of(base_crow, 8)
            pltpu.make_async_copy(ctrl_hbm.at[pl.ds(start, CR)],
                                  buf, sem).wait()

        def out_copy(c, buf, sem):
            start = pl.multiple_of(base_orow + c * OR_, 8)
            pltpu.async_copy(buf, out_hbm.at[pl.ds(start, OR_)], sem)

        def out_wait(buf, sem):
            start = pl.multiple_of(base_orow, 8)
            pltpu.make_async_copy(buf, out_hbm.at[pl.ds(start, OR_)],
                                  sem).wait()

        in_copy(0, cb0, si0)

        def pair(i, carry):
            c0 = 2 * i

            @pl.when(c0 + 1 < nchunks)
            def _():
                in_copy(c0 + 1, cb1, si1)

            in_wait(cb0, si0)

            @pl.when(i > 0)
            def _():
                out_wait(ob0, so0)

            compute_chunk(cb0, ob0)
            out_copy(c0, ob0, so0)

            @pl.when(c0 + 2 < nchunks)
            def _():
                in_copy(c0 + 2, cb0, si0)

            in_wait(cb1, si1)

            @pl.when(i > 0)
            def _():
                out_wait(ob1, so1)

            compute_chunk(cb1, ob1)
            out_copy(c0 + 1, ob1, so1)
            return carry

        lax.fori_loop(0, nchunks // 2, pair, 0)
        out_wait(ob0, so0)
        out_wait(ob1, so1)

    out = sc_kernel(ctrl_flat, Nu, uspan)
    return out.reshape(B, OUT, _DIM)


# speedup vs baseline: 17.4377x; 15.8291x over previous
import functools

import jax
import jax.numpy as jnp
from jax import lax
from jax.experimental import pallas as pl
from jax.experimental.pallas import tpu as pltpu
from jax.experimental.pallas import tpu_sc as plsc

_P = 3
_DIM = 3
_L = 16
_S = 8


def kernel(ctrl_pts, Nu, uspan):
    B, K, D1 = ctrl_pts.shape
    OUT = uspan.shape[0]
    KD = K * D1
    OD = OUT * _DIM
    info = plsc.get_sparse_core_info()
    NC = info.num_cores
    NW = NC * info.num_subcores
    rows_per = B // NW
    NT = OUT // _L
    OW = _S * OD
    nchunks = rows_per // _S

    ctrl_flat = ctrl_pts.reshape(B, KD)

    mesh = plsc.VectorSubcoreMesh(core_axis_name="c", subcore_axis_name="s")

    @functools.partial(
        pl.kernel,
        mesh=mesh,
        compiler_params=pltpu.CompilerParams(needs_layout_passes=False,
                                             use_tc_tiling_on_sc=False),
        out_type=jax.ShapeDtypeStruct((B, OD), jnp.float32),
        scratch_types=[
            pltpu.VMEM((_S * KD,), jnp.float32),
            pltpu.VMEM((_S * KD,), jnp.float32),
            pltpu.VMEM((_S, OD), jnp.float32),
            pltpu.VMEM((_S, OD), jnp.float32),
            pltpu.VMEM((OUT,), jnp.int32),
            pltpu.VMEM((OUT, _P + 1), jnp.float32),
            pltpu.VMEM((OUT,), jnp.int32),
            pltpu.VMEM((OUT,), jnp.int32),
            pltpu.VMEM(((_P + 1) * OUT,), jnp.float32),
            pltpu.SemaphoreType.DMA,
            pltpu.SemaphoreType.DMA,
            pltpu.SemaphoreType.DMA,
            pltpu.SemaphoreType.DMA,
        ],
    )
    def sc_kernel(ctrl_hbm, nu_hbm, usp_hbm, out_hbm,
                  cb0, cb1, ob0, ob1, usp_v, nu_v, g_v, uu_v, nuw_v,
                  si0, si1, so0, so1):
        wid = lax.axis_index("s") * NC + lax.axis_index("c")
        base_row = wid * rows_per

        pltpu.sync_copy(usp_hbm, usp_v)
        pltpu.sync_copy(nu_hbm, nu_v)

        lanes = lax.iota(jnp.int32, _L)

        def build(t, carry):
            u0 = t * _L
            usp = usp_v[pl.ds(u0, _L)]
            g_v[pl.ds(u0, _L)] = (usp - _P) * D1
            uu_v[pl.ds(u0, _L)] = (lanes + u0) * _DIM
            for p in range(_P + 1):
                psplat = jnp.full((_L,), p, jnp.int32)
                nuw_v[pl.ds(p * OUT + u0, _L)] = plsc.load_gather(
                    nu_v, [lanes + u0, psplat])
            return carry

        lax.fori_loop(0, NT, build, 0)

        def compute_chunk(cbuf, obuf):
            def grp(t, carry):
                u0 = t * _L
                g4 = g_v[pl.ds(u0, _L)]
                uu = uu_v[pl.ds(u0, _L)]
                nus = [nuw_v[pl.ds(p * OUT + u0, _L)] for p in range(_P + 1)]
                nj = (_P + 1) * D1
                win = [g4 + j for j in range(nj)]
                for r in range(_S):
                    acc = [None] * D1
                    for j in range(nj):
                        p, d = j // D1, j % D1
                        w = plsc.load_gather(cbuf, [win[j] + r * KD])
                        term = w * nus[p]
                        acc[d] = term if acc[d] is None else acc[d] + term
                    inv = 1.0 / acc[_P]
                    rsplat = jnp.full((_L,), r, jnp.int32)
                    for d in range(_DIM):
                        plsc.store_scatter(obuf, [rsplat, uu + d],
                                           acc[d] * inv)
                return carry

            lax.fori_loop(0, NT, grp, 0)

        def in_copy(c, buf, sem):
            r0 = base_row + c * _S
            for r in range(_S):
                pltpu.async_copy(ctrl_hbm.at[r0 + r],
                                 buf.at[pl.ds(r * KD, KD)], sem)

        def in_wait(buf, sem):
            for r in range(_S):
                pltpu.make_async_copy(ctrl_hbm.at[base_row],
                                      buf.at[pl.ds(r * KD, KD)], sem).wait()

        def out_copy(c, buf, sem):
            pltpu.async_copy(buf, out_hbm.at[pl.ds(base_row + c * _S, _S)],
                             sem)

        def out_wait(buf, sem):
            pltpu.make_async_copy(buf, out_hbm.at[pl.ds(base_row, _S)],
                                  sem).wait()

        in_copy(0, cb0, si0)

        def pair(i, carry):
            c0 = 2 * i

            @pl.when(c0 + 1 < nchunks)
            def _():
                in_copy(c0 + 1, cb1, si1)

            in_wait(cb0, si0)

            @pl.when(i > 0)
            def _():
                out_wait(ob0, so0)

            compute_chunk(cb0, ob0)
            out_copy(c0, ob0, so0)

            @pl.when(c0 + 2 < nchunks)
            def _():
                in_copy(c0 + 2, cb0, si0)

            in_wait(cb1, si1)

            @pl.when(i > 0)
            def _():
                out_wait(ob1, so1)

            compute_chunk(cb1, ob1)
            out_copy(c0 + 1, ob1, so1)
            return carry

        lax.fori_loop(0, nchunks // 2, pair, 0)
        out_wait(ob0, so0)
        out_wait(ob1, so1)

    out = sc_kernel(ctrl_flat, Nu, uspan)
    return out.reshape(B, OUT, _DIM)


# speedup vs baseline: 20.4159x; 1.1708x over previous
import functools

import jax
import jax.numpy as jnp
from jax import lax
from jax.experimental import pallas as pl
from jax.experimental.pallas import tpu as pltpu
from jax.experimental.pallas import tpu_sc as plsc

_P = 3
_DIM = 3
_L = 16


_S = 8


def kernel(ctrl_pts, Nu, uspan):
    B, K, D1 = ctrl_pts.shape
    OUT = uspan.shape[0]
    KD = K * D1
    OD = OUT * _DIM
    info = plsc.get_sparse_core_info()
    NC = info.num_cores
    NW = NC * info.num_subcores
    rows_per = B // NW
    NT = OUT // _L
    CW = _S * KD
    OW = _S * OD
    nchunks = rows_per // _S

    ctrl_flat = ctrl_pts.reshape(B, KD)
    nu_flat = Nu.reshape(OUT * (_P + 1))

    mesh = plsc.VectorSubcoreMesh(core_axis_name="c", subcore_axis_name="s")

    @functools.partial(
        pl.kernel,
        mesh=mesh,
        compiler_params=pltpu.CompilerParams(needs_layout_passes=False),
        out_type=jax.ShapeDtypeStruct((B, OD), jnp.float32),
        scratch_types=[
            pltpu.VMEM((CW,), jnp.float32),
            pltpu.VMEM((CW,), jnp.float32),
            pltpu.VMEM((OW,), jnp.float32),
            pltpu.VMEM((OW,), jnp.float32),
            pltpu.VMEM((OUT,), jnp.int32),
            pltpu.VMEM((OUT * (_P + 1),), jnp.float32),
            pltpu.VMEM((OUT,), jnp.int32),
            pltpu.VMEM((OUT,), jnp.int32),
            pltpu.VMEM(((_P + 1) * OUT,), jnp.float32),
            pltpu.SemaphoreType.DMA,
            pltpu.SemaphoreType.DMA,
            pltpu.SemaphoreType.DMA,
            pltpu.SemaphoreType.DMA,
        ],
    )
    def sc_kernel(ctrl_hbm, nu_hbm, usp_hbm, out_hbm,
                  cb0, cb1, ob0, ob1, usp_v, nu_v, g4_v, b3_v, nuw_v,
                  si0, si1, so0, so1):
        wid = lax.axis_index("s") * NC + lax.axis_index("c")
        base_row = wid * rows_per

        pltpu.sync_copy(usp_hbm, usp_v)
        pltpu.sync_copy(nu_hbm, nu_v)

        lanes = lax.iota(jnp.int32, _L)

        def build(t, carry):
            u0 = t * _L
            usp = usp_v[pl.ds(u0, _L)]
            g4_v[pl.ds(u0, _L)] = (usp - _P) * (_P + 1)
            b3_v[pl.ds(u0, _L)] = (lanes + u0) * _DIM
            for p in range(_P + 1):
                idx = (lanes + u0) * (_P + 1) + p
                nuw_v[pl.ds(p * OUT + u0, _L)] = plsc.load_gather(nu_v, [idx])
            return carry

        lax.fori_loop(0, NT, build, 0)

        def compute_chunk(cbuf, obuf):
            def grp(t, carry):
                u0 = t * _L
                g4 = g4_v[pl.ds(u0, _L)]
                b3 = b3_v[pl.ds(u0, _L)]
                nus = [nuw_v[pl.ds(p * OUT + u0, _L)] for p in range(_P + 1)]
                nj = (_P + 1) * (_P + 1)
                win = [g4 + j for j in range(nj)]
                bd = [b3 + d for d in range(_DIM)]
                for r in range(_S):
                    acc = [None] * (_P + 1)
                    for j in range(nj):
                        p, d = j // (_P + 1), j % (_P + 1)
                        w = plsc.load_gather(cbuf, [win[j] + r * KD])
                        term = w * nus[p]
                        acc[d] = term if acc[d] is None else acc[d] + term
                    inv = 1.0 / acc[_P]
                    for d in range(_DIM):
                        plsc.store_scatter(obuf, [bd[d] + r * OD],
                                           acc[d] * inv)
                return carry

            lax.fori_loop(0, NT, grp, 0)

        def in_copy(c, buf, sem):
            r0 = base_row + c * _S
            for r in range(_S):
                pltpu.async_copy(ctrl_hbm.at[r0 + r],
                                 buf.at[pl.ds(r * KD, KD)], sem)

        def in_wait(buf, sem):
            for r in range(_S):
                pltpu.make_async_copy(ctrl_hbm.at[base_row],
                                      buf.at[pl.ds(r * KD, KD)], sem).wait()

        def out_copy(c, buf, sem):
            r0 = base_row + c * _S
            for r in range(_S):
                pltpu.async_copy(buf.at[pl.ds(r * OD, OD)],
                                 out_hbm.at[r0 + r], sem)

        def out_wait(buf, sem):
            for r in range(_S):
                pltpu.make_async_copy(buf.at[pl.ds(r * OD, OD)],
                                      out_hbm.at[base_row], sem).wait()

        in_copy(0, cb0, si0)

        def pair(i, carry):
            c0 = 2 * i

            @pl.when(c0 + 1 < nchunks)
            def _():
                in_copy(c0 + 1, cb1, si1)

            in_wait(cb0, si0)

            @pl.when(i > 0)
            def _():
                out_wait(ob0, so0)

            compute_chunk(cb0, ob0)
            out_copy(c0, ob0, so0)

            @pl.when(c0 + 2 < nchunks)
            def _():
                in_copy(c0 + 2, cb0, si0)

            in_wait(cb1, si1)

            @pl.when(i > 0)
            def _():
                out_wait(ob1, so1)

            compute_chunk(cb1, ob1)
            out_copy(c0 + 1, ob1, so1)
            return carry

        lax.fori_loop(0, nchunks // 2, pair, 0)
        out_wait(ob0, so0)
        out_wait(ob1, so1)

    out = sc_kernel(ctrl_flat, nu_flat, uspan)
    return out.reshape(B, OUT, _DIM)
